# Initial kernel scaffold; baseline (speedup 1.0000x reference)
import jax
import jax.numpy as jnp
from jax.experimental import pallas as pl


def kernel(emb_table, inputs):
    raise NotImplementedError("write your pallas kernel here")


# speedup vs baseline: 2.5534x; 2.5534x over previous
import functools

import jax
import jax.numpy as jnp
from jax import lax
from jax.experimental import pallas as pl
from jax.experimental.pallas import tpu as pltpu
from jax.experimental.pallas import tpu_sc as plsc

VOCAB_ = 1000000
DIM = 32
BATCH_ = 16384
HIST = 50

NC = 2
NS = 16
NW = NC * NS
B_PER_W = BATCH_ // NW
CHUNK = 64
ROWS = CHUNK * HIST
ROUNDS = B_PER_W // CHUNK


def _sc_kernel(table_hbm, idx_hbm, out_hbm, idx_v, rows_v, out_v, sem):
    wid = lax.axis_index("s") * NC + lax.axis_index("c")
    zero = jnp.zeros((16,), jnp.float32)
    inv = jnp.float32(1.0 / HIST)

    def round_body(r, carry):
        base_b = wid * B_PER_W + r * CHUNK
        pltpu.sync_copy(idx_hbm.at[pl.ds(base_b * HIST, ROWS)], idx_v)
        pltpu.async_copy(table_hbm.at[idx_v], rows_v, sem).wait()

        def batch_body(b, c):
            def hist_body(h, accs):
                a0, a1 = accs
                row = b * HIST + h
                return (a0 + rows_v[row, pl.ds(0, 16)],
                        a1 + rows_v[row, pl.ds(16, 16)])

            a0, a1 = lax.fori_loop(0, HIST, hist_body, (zero, zero))
            out_v[b, pl.ds(0, 16)] = a0 * inv
            out_v[b, pl.ds(16, 16)] = a1 * inv
            return c

        lax.fori_loop(0, CHUNK, batch_body, 0)
        pltpu.sync_copy(out_v, out_hbm.at[pl.ds(base_b, CHUNK)])
        return carry

    lax.fori_loop(0, ROUNDS, round_body, 0)


@jax.jit
def _pooled_lookup(emb_table, idx_flat):
    mesh = plsc.VectorSubcoreMesh(core_axis_name="c", subcore_axis_name="s")
    f = functools.partial(
        pl.kernel,
        mesh=mesh,
        out_type=jax.ShapeDtypeStruct((BATCH_, DIM), jnp.float32),
        scratch_types=[
            pltpu.VMEM((ROWS,), jnp.int32),
            pltpu.VMEM((ROWS, DIM), jnp.float32),
            pltpu.VMEM((CHUNK, DIM), jnp.float32),
            pltpu.SemaphoreType.DMA,
        ],
        compiler_params=pltpu.CompilerParams(use_tc_tiling_on_sc=False),
    )(_sc_kernel)
    return f(emb_table, idx_flat)


def kernel(emb_table, inputs):
    return _pooled_lookup(emb_table, inputs.reshape(-1))


# speedup vs baseline: 2.9343x; 1.1492x over previous
import functools

import jax
import jax.numpy as jnp
from jax import lax
from jax.experimental import pallas as pl
from jax.experimental.pallas import tpu as pltpu
from jax.experimental.pallas import tpu_sc as plsc

DIM = 32
BATCH_ = 16384
HIST = 50

NC = 2
NS = 16
NW = NC * NS
B_PER_W = BATCH_ // NW
CHUNK = 32
ROWS = CHUNK * HIST
ROUNDS = B_PER_W // CHUNK


def _sc_kernel(table_hbm, idx_hbm, out_hbm,
               idx0, idx1, rows0, rows1, out_v, sem0, sem1):
    wid = lax.axis_index("s") * NC + lax.axis_index("c")
    base_b0 = wid * B_PER_W
    zero = jnp.zeros((16,), jnp.float32)
    inv = jnp.float32(1.0 / HIST)
    idx_b = (idx0, idx1)
    rows_b = (rows0, rows1)
    sems = (sem0, sem1)

    def start(r, p):
        pltpu.sync_copy(idx_hbm.at[pl.ds((base_b0 + r * CHUNK) * HIST, ROWS)],
                        idx_b[p])
        pltpu.async_copy(table_hbm.at[idx_b[p]], rows_b[p], sems[p])

    def process(r, p):
        rows_v = rows_b[p]

        def batch_body(b, c):
            base_row = b * HIST
            a0 = zero
            a1 = zero
            c0 = zero
            c1 = zero
            for h in range(0, HIST, 2):
                a0 = a0 + rows_v[base_row + h, pl.ds(0, 16)]
                a1 = a1 + rows_v[base_row + h, pl.ds(16, 16)]
                c0 = c0 + rows_v[base_row + h + 1, pl.ds(0, 16)]
                c1 = c1 + rows_v[base_row + h + 1, pl.ds(16, 16)]
            out_v[b, pl.ds(0, 16)] = (a0 + c0) * inv
            out_v[b, pl.ds(16, 16)] = (a1 + c1) * inv
            return c

        lax.fori_loop(0, CHUNK, batch_body, 0)
        pltpu.sync_copy(out_v, out_hbm.at[pl.ds(base_b0 + r * CHUNK, CHUNK)])

    start(0, 0)

    def outer(rr, carry):
        for p in (0, 1):
            r = rr * 2 + p
            nxt = (p + 1) % 2

            @pl.when(r + 1 < ROUNDS)
            def _():
                start(r + 1, nxt)

            pltpu.make_async_copy(table_hbm.at[idx_b[p]], rows_b[p],
                                  sems[p]).wait()
            process(r, p)
        return carry

    lax.fori_loop(0, ROUNDS // 2, outer, 0)


@jax.jit
def _pooled_lookup(emb_table, idx_flat):
    mesh = plsc.VectorSubcoreMesh(core_axis_name="c", subcore_axis_name="s")
    f = functools.partial(
        pl.kernel,
        mesh=mesh,
        out_type=jax.ShapeDtypeStruct((BATCH_, DIM), jnp.float32),
        scratch_types=[
            pltpu.VMEM((ROWS,), jnp.int32),
            pltpu.VMEM((ROWS,), jnp.int32),
            pltpu.VMEM((ROWS, DIM), jnp.float32),
            pltpu.VMEM((ROWS, DIM), jnp.float32),
            pltpu.VMEM((CHUNK, DIM), jnp.float32),
            pltpu.SemaphoreType.DMA,
            pltpu.SemaphoreType.DMA,
        ],
        compiler_params=pltpu.CompilerParams(use_tc_tiling_on_sc=False),
    )(_sc_kernel)
    return f(emb_table, idx_flat)


def kernel(emb_table, inputs):
    return _pooled_lookup(emb_table, inputs.reshape(-1))
